# Initial kernel scaffold; baseline (speedup 1.0000x reference)
import jax
import jax.numpy as jnp
from jax.experimental import pallas as pl


def kernel(att_feat, aud_feat, att_heatmaps, av_heatmaps):
    raise NotImplementedError("write your pallas kernel here")


# speedup vs baseline: 2.0078x; 2.0078x over previous
import functools

import jax
import jax.numpy as jnp
import numpy as np
from jax import lax
from jax.experimental import pallas as pl
from jax.experimental.pallas import tpu as pltpu
from jax.experimental.pallas import tpu_sc as plsc

B, C, K = 64, 512, 1024
HW = 1024
FG = 128
BG = 128
INT_MIN = np.int32(-(2 ** 31))
POS_MASK = np.int32(2 ** 31 - 1)

NUM_SC_CORES = 2
NUM_SUBCORES = 16
NUM_WORKERS = NUM_SC_CORES * NUM_SUBCORES
B_PER_W = B // NUM_WORKERS
GATHER_ROWS = 32


def _bit(b):
    return INT_MIN if b == 31 else np.int32(1 << b)


def _low_mask(b):
    return np.int32((1 << b) - 1)


def _sim_topk_body(att_ref, aud_ref, keys_ref, thr_ref, ngt_ref, pos_ref,
                   neg_ref):
    f = att_ref[0]
    a = aud_ref[0]
    dot = jnp.dot(a, f, preferred_element_type=jnp.float32)
    ss = jnp.sum(f * f, axis=0, keepdims=True)
    sim = dot / jnp.maximum(jnp.sqrt(ss), 1e-12)

    ibits = lax.bitcast_convert_type(sim, jnp.int32)
    key = jnp.where(ibits >= 0, ibits, ibits ^ POS_MASK)

    prefix = jnp.zeros((1, 1), jnp.int32)
    for b in range(31, -1, -1):
        trial = prefix | _bit(b)
        cnt = jnp.sum((key >= (trial ^ INT_MIN)).astype(jnp.int32))
        prefix = jnp.where(cnt >= FG, trial, prefix)
    s_top = prefix ^ INT_MIN

    prefixb = jnp.zeros((1, 1), jnp.int32)
    for b in range(31, -1, -1):
        trial = prefixb | _low_mask(b)
        cnt = jnp.sum((key <= (trial ^ INT_MIN)).astype(jnp.int32))
        prefixb = jnp.where(cnt >= BG, prefixb, prefixb | _bit(b))
    s_bot = prefixb ^ INT_MIN

    gt = key > s_top
    n_gt = jnp.sum(gt.astype(jnp.int32))
    sum_gt = jnp.sum(jnp.where(gt, sim, 0.0))
    v_top = lax.bitcast_convert_type(
        jnp.where(s_top >= 0, s_top, s_top ^ POS_MASK), jnp.float32)[0, 0]
    pos = (sum_gt + (FG - n_gt).astype(jnp.float32) * v_top) / FG

    lt = key < s_bot
    n_lt = jnp.sum(lt.astype(jnp.int32))
    sum_lt = jnp.sum(jnp.where(lt, sim, 0.0))
    v_bot = lax.bitcast_convert_type(
        jnp.where(s_bot >= 0, s_bot, s_bot ^ POS_MASK), jnp.float32)[0, 0]
    neg = (sum_lt + (BG - n_lt).astype(jnp.float32) * v_bot) / BG

    keys_ref[...] = key.reshape(1, 1, K)
    thr_ref[...] = jnp.broadcast_to(s_top.reshape(1, 1, 1), (1, 1, 16))
    ngt_ref[...] = jnp.broadcast_to(n_gt.reshape(1, 1, 1), (1, 1, 16))
    pos_ref[...] = jnp.full((1, 1, 16), pos, jnp.float32)
    neg_ref[...] = jnp.full((1, 1, 16), neg, jnp.float32)


def _run_stage_a(att_feat, aud_feat):
    aud3 = aud_feat.reshape(B, 1, C)
    return pl.pallas_call(
        _sim_topk_body,
        grid=(B,),
        in_specs=[
            pl.BlockSpec((1, C, K), lambda b: (b, 0, 0)),
            pl.BlockSpec((1, 1, C), lambda b: (b, 0, 0)),
        ],
        out_specs=[
            pl.BlockSpec((1, 1, K), lambda b: (b, 0, 0)),
            pl.BlockSpec((1, 1, 16), lambda b: (b, 0, 0)),
            pl.BlockSpec((1, 1, 16), lambda b: (b, 0, 0)),
            pl.BlockSpec((1, 1, 16), lambda b: (b, 0, 0)),
            pl.BlockSpec((1, 1, 16), lambda b: (b, 0, 0)),
        ],
        out_shape=[
            jax.ShapeDtypeStruct((B, 1, K), jnp.int32),
            jax.ShapeDtypeStruct((B, 1, 16), jnp.int32),
            jax.ShapeDtypeStruct((B, 1, 16), jnp.int32),
            jax.ShapeDtypeStruct((B, 1, 16), jnp.float32),
            jax.ShapeDtypeStruct((B, 1, 16), jnp.float32),
        ],
    )(att_feat, aud3)


def _sc_gather_body(keys_hbm, thr_hbm, ngt_hbm, table_hbm, out_hbm,
                    keys_v, thr_v, ngt_v, idx_v, buf_v, acc_v, sem):
    cid = lax.axis_index("c")
    sid = lax.axis_index("s")
    wid = cid * NUM_SUBCORES + sid
    iota = lax.iota(jnp.int32, 16)
    zeros16i = jnp.zeros((16,), jnp.int32)

    for i in range(B_PER_W):
        b = wid * B_PER_W + i
        pltpu.sync_copy(keys_hbm.at[b], keys_v)
        pltpu.sync_copy(thr_hbm.at[b], thr_v)
        pltpu.sync_copy(ngt_hbm.at[b], ngt_v)
        thr = thr_v[...]
        ngt = ngt_v[...]

        cgt = zeros16i
        ceq = zeros16i
        for j in range(K // 16):
            kvec = keys_v[pl.ds(j * 16, 16)]
            ids = iota + (b * K + j * 16)
            m_gt = kvec > thr
            incl = plsc.cumsum(m_gt.astype(jnp.int32))
            plsc.store_scatter(idx_v, [cgt + incl - 1], ids, mask=m_gt)
            m_eq = kvec == thr
            incl_e = plsc.cumsum(m_eq.astype(jnp.int32))
            pos_eq = ngt + ceq + incl_e - 1
            m_sel = jnp.logical_and(m_eq, pos_eq < FG)
            plsc.store_scatter(idx_v, [pos_eq], ids, mask=m_sel)
            cgt = cgt + plsc.all_reduce_population_count(m_gt)
            ceq = ceq + plsc.all_reduce_population_count(m_eq)

        for g in range(FG // GATHER_ROWS):
            pltpu.async_copy(
                table_hbm.at[idx_v.at[pl.ds(g * GATHER_ROWS, GATHER_ROWS)]],
                buf_v, sem).wait()

            def acc_block(c, carry, first=(g == 0)):
                off = pl.multiple_of(c * 16, 16)
                s = buf_v[0, pl.ds(off, 16)]
                for r in range(1, GATHER_ROWS):
                    s = s + buf_v[r, pl.ds(off, 16)]
                if first:
                    acc_v[pl.ds(off, 16)] = s
                else:
                    plsc.addupdate(acc_v.at[pl.ds(off, 16)], s)
                return carry

            lax.fori_loop(0, HW // 16, acc_block, 0)

        pltpu.sync_copy(acc_v, out_hbm.at[b])


def _run_stage_b(keys, thr, ngt, table):
    mesh = plsc.VectorSubcoreMesh(core_axis_name="c", subcore_axis_name="s")
    fn = functools.partial(
        pl.kernel,
        out_type=jax.ShapeDtypeStruct((B, HW), jnp.float32),
        mesh=mesh,
        compiler_params=pltpu.CompilerParams(needs_layout_passes=False),
        scratch_types=[
            pltpu.VMEM((K,), jnp.int32),
            pltpu.VMEM((16,), jnp.int32),
            pltpu.VMEM((16,), jnp.int32),
            pltpu.VMEM((FG,), jnp.int32),
            pltpu.VMEM((GATHER_ROWS, HW), jnp.float32),
            pltpu.VMEM((HW,), jnp.float32),
            pltpu.SemaphoreType.DMA,
        ],
    )(_sc_gather_body)
    return fn(keys, thr, ngt, table)


def _loss_body(pos_ref, neg_ref, comb_ref, av_ref, dis_ref, div_ref):
    p = pos_ref[:, 0:1]
    n = neg_ref[:, 0:1]
    m = jnp.maximum(p, n)
    lse = m + jnp.log(jnp.exp(p - m) + jnp.exp(n - m))
    dis_ref[...] = jnp.mean(lse - p).reshape(1, 1)

    c = comb_ref[...] * (1.0 / FG)
    a = av_ref[...]
    cm = jnp.max(c, axis=1, keepdims=True)
    ce = jnp.exp(c - cm)
    cz = jnp.sum(ce, axis=1, keepdims=True)
    att = ce / cz
    log_att = (c - cm) - jnp.log(cz)
    am = jnp.max(a, axis=1, keepdims=True)
    ae = jnp.exp(a - am)
    az = jnp.sum(ae, axis=1, keepdims=True)
    avd = ae / az
    log_av = (a - am) - jnp.log(az)
    logm = jnp.log(0.5 * (att + avd))
    div = (jnp.sum(att * (log_att - logm)) +
           jnp.sum(avd * (log_av - logm))) / (2.0 * B)
    div_ref[...] = div.reshape(1, 1)


def _run_stage_c(pos, neg, comb, av):
    return pl.pallas_call(
        _loss_body,
        out_shape=[
            jax.ShapeDtypeStruct((1, 1), jnp.float32),
            jax.ShapeDtypeStruct((1, 1), jnp.float32),
        ],
    )(pos, neg, comb, av)


def kernel(att_feat, aud_feat, att_heatmaps, av_heatmaps):
    keys, thr, ngt, pos, neg = _run_stage_a(att_feat, aud_feat)
    table = att_heatmaps.reshape(B * K, HW)
    comb = _run_stage_b(keys.reshape(B, K), thr.reshape(B, 16),
                        ngt.reshape(B, 16), table)
    dis, div = _run_stage_c(pos.reshape(B, 16), neg.reshape(B, 16),
                            comb, av_heatmaps.reshape(B, HW))
    return dis.reshape(()), div.reshape(())


# speedup vs baseline: 11.1671x; 5.5619x over previous
import functools

import jax
import jax.numpy as jnp
import numpy as np
from jax import lax
from jax.experimental import pallas as pl
from jax.experimental.pallas import tpu as pltpu
from jax.experimental.pallas import tpu_sc as plsc

B, C, K = 64, 512, 1024
H, W = 32, 32
HW = H * W
FG = 128
BG = 128
INT_MIN = np.int32(-(2 ** 31))
POS_MASK = np.int32(2 ** 31 - 1)

NUM_SC_CORES = 2
NUM_SUBCORES = 16
NUM_WORKERS = NUM_SC_CORES * NUM_SUBCORES
B_PER_W = B // NUM_WORKERS


def _bit(b):
    return INT_MIN if b == 31 else np.int32(1 << b)


def _low_mask(b):
    return np.int32((1 << b) - 1)


A1_ROWS = 8


def _sim_body(att_ref, aud_ref, sim_ref):
    for i in range(A1_ROWS):
        f = att_ref[i]
        a = aud_ref[i]
        dot = jnp.sum(f * a, axis=0, keepdims=True)
        ss = jnp.sum(f * f, axis=0, keepdims=True)
        sim_ref[pl.ds(i, 1), :] = dot / jnp.maximum(jnp.sqrt(ss), 1e-12)


def _run_stage_a1(att_feat, aud_feat):
    aud3 = aud_feat.reshape(B, C, 1)
    return pl.pallas_call(
        _sim_body,
        grid=(B // A1_ROWS,),
        in_specs=[
            pl.BlockSpec((A1_ROWS, C, K), lambda b: (b, 0, 0)),
            pl.BlockSpec((A1_ROWS, C, 1), lambda b: (b, 0, 0)),
        ],
        out_specs=pl.BlockSpec((A1_ROWS, K), lambda b: (b, 0)),
        out_shape=jax.ShapeDtypeStruct((B, K), jnp.float32),
    )(att_feat, aud3)


def _topk_body(sim_ref, thr_ref, ngt_ref, pos_ref, neg_ref):
    sim = sim_ref[...]
    ibits = lax.bitcast_convert_type(sim, jnp.int32)
    key = jnp.where(ibits >= 0, ibits, ibits ^ POS_MASK)

    prefix = jnp.zeros((B, 1), jnp.int32)
    for b in range(31, -1, -1):
        trial = prefix | _bit(b)
        cnt = jnp.sum((key >= (trial ^ INT_MIN)).astype(jnp.float32),
                      axis=1, keepdims=True)
        prefix = jnp.where(cnt >= float(FG), trial, prefix)
    s_top = prefix ^ INT_MIN

    prefixb = jnp.zeros((B, 1), jnp.int32)
    for b in range(31, -1, -1):
        trial = prefixb | _low_mask(b)
        cnt = jnp.sum((key <= (trial ^ INT_MIN)).astype(jnp.float32),
                      axis=1, keepdims=True)
        prefixb = jnp.where(cnt >= float(BG), prefixb, prefixb | _bit(b))
    s_bot = prefixb ^ INT_MIN

    gt = key > s_top
    n_gt = jnp.sum(gt.astype(jnp.float32), axis=1, keepdims=True)
    sum_gt = jnp.sum(jnp.where(gt, sim, 0.0), axis=1, keepdims=True)
    v_top = lax.bitcast_convert_type(
        jnp.where(s_top >= 0, s_top, s_top ^ POS_MASK), jnp.float32)
    pos = (sum_gt + (FG - n_gt) * v_top) / FG

    lt = key < s_bot
    n_lt = jnp.sum(lt.astype(jnp.float32), axis=1, keepdims=True)
    sum_lt = jnp.sum(jnp.where(lt, sim, 0.0), axis=1, keepdims=True)
    v_bot = lax.bitcast_convert_type(
        jnp.where(s_bot >= 0, s_bot, s_bot ^ POS_MASK), jnp.float32)
    neg = (sum_lt + (BG - n_lt) * v_bot) / BG

    thr_ref[...] = jnp.broadcast_to(v_top, (B, 16))
    ngt_ref[...] = jnp.broadcast_to(n_gt.astype(jnp.int32), (B, 16))
    pos_ref[...] = jnp.broadcast_to(pos, (B, 16))
    neg_ref[...] = jnp.broadcast_to(neg, (B, 16))


def _run_stage_a2(sim):
    return pl.pallas_call(
        _topk_body,
        out_shape=[
            jax.ShapeDtypeStruct((B, 16), jnp.float32),
            jax.ShapeDtypeStruct((B, 16), jnp.int32),
            jax.ShapeDtypeStruct((B, 16), jnp.float32),
            jax.ShapeDtypeStruct((B, 16), jnp.float32),
        ],
    )(sim)


def _sc_mask_body(sim_hbm, thr_hbm, ngt_hbm, out_hbm,
                  sim_v, thr_v, ngt_v, mask_v, sem):
    del sem
    cid = lax.axis_index("c")
    sid = lax.axis_index("s")
    wid = cid * NUM_SUBCORES + sid
    zeros16i = jnp.zeros((16,), jnp.int32)
    ones16f = jnp.full((16,), 1.0, jnp.float32)
    zeros16f = jnp.zeros((16,), jnp.float32)

    for i in range(B_PER_W):
        b = wid * B_PER_W + i
        pltpu.sync_copy(sim_hbm.at[b], sim_v)
        pltpu.sync_copy(thr_hbm.at[b], thr_v)
        pltpu.sync_copy(ngt_hbm.at[b], ngt_v)
        thr = thr_v[...]
        ngt = ngt_v[...]

        ceq = zeros16i
        for j in range(K // 16):
            svec = sim_v[pl.ds(j * 16, 16)]
            m_gt = svec > thr
            m_eq = svec == thr
            incl_e = plsc.cumsum(m_eq.astype(jnp.int32))
            tie_rank = ngt + ceq + incl_e - 1
            m_sel = jnp.logical_or(
                m_gt, jnp.logical_and(m_eq, tie_rank < FG))
            mask_v[pl.ds(j * 16, 16)] = jnp.where(m_sel, ones16f, zeros16f)
            ceq = ceq + plsc.all_reduce_population_count(m_eq)

        pltpu.sync_copy(mask_v, out_hbm.at[b])


def _run_stage_b1(sim, thr, ngt):
    mesh = plsc.VectorSubcoreMesh(core_axis_name="c", subcore_axis_name="s")
    fn = functools.partial(
        pl.kernel,
        out_type=jax.ShapeDtypeStruct((B, K), jnp.float32),
        mesh=mesh,
        compiler_params=pltpu.CompilerParams(needs_layout_passes=False),
        scratch_types=[
            pltpu.VMEM((K,), jnp.float32),
            pltpu.VMEM((16,), jnp.float32),
            pltpu.VMEM((16,), jnp.int32),
            pltpu.VMEM((K,), jnp.float32),
            pltpu.SemaphoreType.DMA,
        ],
    )(_sc_mask_body)
    return fn(sim, thr, ngt)


def _hm_reduce_body(hm_ref, mask_ref, out_ref):
    hm = hm_ref[0].reshape(HW, K)
    m = mask_ref[...].reshape(1, K)
    comb = lax.dot_general(m, hm, (((1,), (1,)), ((), ())),
                           preferred_element_type=jnp.float32)
    out_ref[...] = comb.reshape(1, 1, HW)


def _run_stage_b2(hm_t, mask):
    return pl.pallas_call(
        _hm_reduce_body,
        grid=(B,),
        in_specs=[
            pl.BlockSpec((1, H, W, K), lambda b: (b, 0, 0, 0)),
            pl.BlockSpec((1, 1, K), lambda b: (b, 0, 0)),
        ],
        out_specs=pl.BlockSpec((1, 1, HW), lambda b: (b, 0, 0)),
        out_shape=jax.ShapeDtypeStruct((B, 1, HW), jnp.float32),
    )(hm_t, mask.reshape(B, 1, K))


def _loss_body(pos_ref, neg_ref, comb_ref, av_ref, dis_ref, div_ref):
    p = pos_ref[:, 0:1]
    n = neg_ref[:, 0:1]
    m = jnp.maximum(p, n)
    lse = m + jnp.log(jnp.exp(p - m) + jnp.exp(n - m))
    dis_ref[...] = jnp.mean(lse - p).reshape(1, 1)

    c = comb_ref[:, 0, :] * (1.0 / FG)
    a = av_ref[...]
    cm = jnp.max(c, axis=1, keepdims=True)
    ce = jnp.exp(c - cm)
    cz = jnp.sum(ce, axis=1, keepdims=True)
    att = ce / cz
    log_att = (c - cm) - jnp.log(cz)
    am = jnp.max(a, axis=1, keepdims=True)
    ae = jnp.exp(a - am)
    az = jnp.sum(ae, axis=1, keepdims=True)
    avd = ae / az
    log_av = (a - am) - jnp.log(az)
    logm = jnp.log(0.5 * (att + avd))
    div = (jnp.sum(att * (log_att - logm)) +
           jnp.sum(avd * (log_av - logm))) / (2.0 * B)
    div_ref[...] = div.reshape(1, 1)


def _run_stage_c(pos, neg, comb, av):
    return pl.pallas_call(
        _loss_body,
        out_shape=[
            jax.ShapeDtypeStruct((1, 1), jnp.float32),
            jax.ShapeDtypeStruct((1, 1), jnp.float32),
        ],
    )(pos, neg, comb, av)


def kernel(att_feat, aud_feat, att_heatmaps, av_heatmaps):
    sim = _run_stage_a1(att_feat, aud_feat)
    thr, ngt, pos, neg = _run_stage_a2(sim)
    mask = _run_stage_b1(sim, thr, ngt)
    hm_t = jnp.transpose(att_heatmaps, (0, 2, 3, 1))
    comb = _run_stage_b2(hm_t, mask)
    dis, div = _run_stage_c(pos, neg, comb, av_heatmaps.reshape(B, HW))
    return dis.reshape(()), div.reshape(())


# speedup vs baseline: 12.2449x; 1.0965x over previous
import functools

import jax
import jax.numpy as jnp
import numpy as np
from jax import lax
from jax.experimental import pallas as pl
from jax.experimental.pallas import tpu as pltpu
from jax.experimental.pallas import tpu_sc as plsc

B, C, K = 64, 512, 1024
H, W = 32, 32
HW = H * W
FG = 128
BG = 128
INT_MIN = np.int32(-(2 ** 31))
POS_MASK = np.int32(2 ** 31 - 1)

NUM_SC_CORES = 2
NUM_SUBCORES = 16
NUM_WORKERS = NUM_SC_CORES * NUM_SUBCORES
B_PER_W = B // NUM_WORKERS


def _bit(b):
    return INT_MIN if b == 31 else np.int32(1 << b)


def _low_mask(b):
    return np.int32((1 << b) - 1)


A1_ROWS = 8


def _sim_body(att_ref, aud_ref, sim_ref):
    for i in range(A1_ROWS):
        f = att_ref[i]
        a = aud_ref[i]
        dot = jnp.sum(f * a, axis=0, keepdims=True)
        ss = jnp.sum(f * f, axis=0, keepdims=True)
        sim_ref[pl.ds(i, 1), :] = dot / jnp.maximum(jnp.sqrt(ss), 1e-12)


def _run_stage_a1(att_feat, aud_feat):
    aud3 = aud_feat.reshape(B, C, 1)
    return pl.pallas_call(
        _sim_body,
        grid=(B // A1_ROWS,),
        in_specs=[
            pl.BlockSpec((A1_ROWS, C, K), lambda b: (b, 0, 0)),
            pl.BlockSpec((A1_ROWS, C, 1), lambda b: (b, 0, 0)),
        ],
        out_specs=pl.BlockSpec((A1_ROWS, K), lambda b: (b, 0)),
        out_shape=jax.ShapeDtypeStruct((B, K), jnp.float32),
    )(att_feat, aud3)


def _topk_body(sim_ref, thr_ref, ngt_ref, pos_ref, neg_ref):
    sim = sim_ref[...]
    ibits = lax.bitcast_convert_type(sim, jnp.int32)
    key = jnp.where(ibits >= 0, ibits, ibits ^ POS_MASK)

    prefix = jnp.zeros((B, 1), jnp.int32)
    for b in range(31, -1, -1):
        trial = prefix | _bit(b)
        cnt = jnp.sum((key >= (trial ^ INT_MIN)).astype(jnp.float32),
                      axis=1, keepdims=True)
        prefix = jnp.where(cnt >= float(FG), trial, prefix)
    s_top = prefix ^ INT_MIN

    prefixb = jnp.zeros((B, 1), jnp.int32)
    for b in range(31, -1, -1):
        trial = prefixb | _low_mask(b)
        cnt = jnp.sum((key <= (trial ^ INT_MIN)).astype(jnp.float32),
                      axis=1, keepdims=True)
        prefixb = jnp.where(cnt >= float(BG), prefixb, prefixb | _bit(b))
    s_bot = prefixb ^ INT_MIN

    gt = key > s_top
    n_gt = jnp.sum(gt.astype(jnp.float32), axis=1, keepdims=True)
    sum_gt = jnp.sum(jnp.where(gt, sim, 0.0), axis=1, keepdims=True)
    v_top = lax.bitcast_convert_type(
        jnp.where(s_top >= 0, s_top, s_top ^ POS_MASK), jnp.float32)
    pos = (sum_gt + (FG - n_gt) * v_top) / FG

    lt = key < s_bot
    n_lt = jnp.sum(lt.astype(jnp.float32), axis=1, keepdims=True)
    sum_lt = jnp.sum(jnp.where(lt, sim, 0.0), axis=1, keepdims=True)
    v_bot = lax.bitcast_convert_type(
        jnp.where(s_bot >= 0, s_bot, s_bot ^ POS_MASK), jnp.float32)
    neg = (sum_lt + (BG - n_lt) * v_bot) / BG

    thr_ref[...] = jnp.broadcast_to(v_top, (B, 16))
    ngt_ref[...] = jnp.broadcast_to(n_gt.astype(jnp.int32), (B, 16))
    pos_ref[...] = jnp.broadcast_to(pos, (B, 16))
    neg_ref[...] = jnp.broadcast_to(neg, (B, 16))


def _run_stage_a2(sim):
    return pl.pallas_call(
        _topk_body,
        out_shape=[
            jax.ShapeDtypeStruct((B, 16), jnp.float32),
            jax.ShapeDtypeStruct((B, 16), jnp.int32),
            jax.ShapeDtypeStruct((B, 16), jnp.float32),
            jax.ShapeDtypeStruct((B, 16), jnp.float32),
        ],
    )(sim)


def _sc_mask_body(sim_hbm, thr_hbm, ngt_hbm, out_hbm,
                  sim_v, thr_v, ngt_v, mask_v, sem):
    del sem
    cid = lax.axis_index("c")
    sid = lax.axis_index("s")
    wid = cid * NUM_SUBCORES + sid
    zeros16i = jnp.zeros((16,), jnp.int32)
    ones16f = jnp.full((16,), 1.0, jnp.float32)
    zeros16f = jnp.zeros((16,), jnp.float32)

    for i in range(B_PER_W):
        b = wid * B_PER_W + i
        pltpu.sync_copy(sim_hbm.at[b], sim_v)
        pltpu.sync_copy(thr_hbm.at[b], thr_v)
        pltpu.sync_copy(ngt_hbm.at[b], ngt_v)
        thr = thr_v[...]
        ngt = ngt_v[...]

        ceq = zeros16i
        for j in range(K // 16):
            svec = sim_v[pl.ds(j * 16, 16)]
            m_gt = svec > thr
            m_eq = svec == thr
            incl_e = plsc.cumsum(m_eq.astype(jnp.int32))
            tie_rank = ngt + ceq + incl_e - 1
            m_sel = jnp.logical_or(
                m_gt, jnp.logical_and(m_eq, tie_rank < FG))
            mask_v[pl.ds(j * 16, 16)] = jnp.where(m_sel, ones16f, zeros16f)
            ceq = ceq + plsc.all_reduce_population_count(m_eq)

        pltpu.sync_copy(mask_v, out_hbm.at[b])


def _run_stage_b1(sim, thr, ngt):
    mesh = plsc.VectorSubcoreMesh(core_axis_name="c", subcore_axis_name="s")
    fn = functools.partial(
        pl.kernel,
        out_type=jax.ShapeDtypeStruct((B, K), jnp.float32),
        mesh=mesh,
        compiler_params=pltpu.CompilerParams(needs_layout_passes=False),
        scratch_types=[
            pltpu.VMEM((K,), jnp.float32),
            pltpu.VMEM((16,), jnp.float32),
            pltpu.VMEM((16,), jnp.int32),
            pltpu.VMEM((K,), jnp.float32),
            pltpu.SemaphoreType.DMA,
        ],
    )(_sc_mask_body)
    return fn(sim, thr, ngt)


B2_ROWS = 4


def _hm_reduce_body(hm_ref, mask_ref, out_ref):
    for i in range(B2_ROWS):
        hm = hm_ref[i].reshape(HW, K)
        m = mask_ref[i]
        comb = lax.dot_general(m, hm, (((1,), (1,)), ((), ())),
                               preferred_element_type=jnp.float32)
        out_ref[pl.ds(i, 1), :, :] = comb.reshape(1, 1, HW)


def _run_stage_b2(hm_t, mask):
    return pl.pallas_call(
        _hm_reduce_body,
        grid=(B // B2_ROWS,),
        in_specs=[
            pl.BlockSpec((B2_ROWS, H, W, K), lambda b: (b, 0, 0, 0)),
            pl.BlockSpec((B2_ROWS, 1, K), lambda b: (b, 0, 0)),
        ],
        out_specs=pl.BlockSpec((B2_ROWS, 1, HW), lambda b: (b, 0, 0)),
        out_shape=jax.ShapeDtypeStruct((B, 1, HW), jnp.float32),
    )(hm_t, mask.reshape(B, 1, K))


def _loss_body(pos_ref, neg_ref, comb_ref, av_ref, dis_ref, div_ref):
    p = pos_ref[:, 0:1]
    n = neg_ref[:, 0:1]
    m = jnp.maximum(p, n)
    lse = m + jnp.log(jnp.exp(p - m) + jnp.exp(n - m))
    dis_ref[...] = jnp.mean(lse - p).reshape(1, 1)

    c = comb_ref[:, 0, :] * (1.0 / FG)
    a = av_ref[...]
    cm = jnp.max(c, axis=1, keepdims=True)
    ce = jnp.exp(c - cm)
    cz = jnp.sum(ce, axis=1, keepdims=True)
    att = ce / cz
    log_att = (c - cm) - jnp.log(cz)
    am = jnp.max(a, axis=1, keepdims=True)
    ae = jnp.exp(a - am)
    az = jnp.sum(ae, axis=1, keepdims=True)
    avd = ae / az
    log_av = (a - am) - jnp.log(az)
    logm = jnp.log(0.5 * (att + avd))
    div = (jnp.sum(att * (log_att - logm)) +
           jnp.sum(avd * (log_av - logm))) / (2.0 * B)
    div_ref[...] = div.reshape(1, 1)


def _run_stage_c(pos, neg, comb, av):
    return pl.pallas_call(
        _loss_body,
        out_shape=[
            jax.ShapeDtypeStruct((1, 1), jnp.float32),
            jax.ShapeDtypeStruct((1, 1), jnp.float32),
        ],
    )(pos, neg, comb, av)


def kernel(att_feat, aud_feat, att_heatmaps, av_heatmaps):
    sim = _run_stage_a1(att_feat, aud_feat)
    thr, ngt, pos, neg = _run_stage_a2(sim)
    mask = _run_stage_b1(sim, thr, ngt)
    hm_t = jnp.transpose(att_heatmaps, (0, 2, 3, 1))
    comb = _run_stage_b2(hm_t, mask)
    dis, div = _run_stage_c(pos, neg, comb, av_heatmaps.reshape(B, HW))
    return dis.reshape(()), div.reshape(())


# speedup vs baseline: 12.5809x; 1.0274x over previous
import functools

import jax
import jax.numpy as jnp
import numpy as np
from jax import lax
from jax.experimental import pallas as pl
from jax.experimental.pallas import tpu as pltpu
from jax.experimental.pallas import tpu_sc as plsc

B, C, K = 64, 512, 1024
H, W = 32, 32
HW = H * W
FG = 128
BG = 128
INT_MIN = np.int32(-(2 ** 31))
POS_MASK = np.int32(2 ** 31 - 1)

NUM_SC_CORES = 2
NUM_SUBCORES = 16
NUM_WORKERS = NUM_SC_CORES * NUM_SUBCORES
B_PER_W = B // NUM_WORKERS


def _bit(b):
    return INT_MIN if b == 31 else np.int32(1 << b)


def _low_mask(b):
    return np.int32((1 << b) - 1)


A1_ROWS = 8


def _sim_topk_body(att_ref, aud_ref, sim_ref, thr_ref, ngt_ref, pos_ref,
                   neg_ref, simacc_ref):
    step = pl.program_id(0)
    for i in range(A1_ROWS):
        f = att_ref[i]
        a = aud_ref[i]
        dot = jnp.sum(f * a, axis=0, keepdims=True)
        ss = jnp.sum(f * f, axis=0, keepdims=True)
        row = dot / jnp.maximum(jnp.sqrt(ss), 1e-12)
        sim_ref[pl.ds(i, 1), :] = row
        simacc_ref[pl.ds(step * A1_ROWS + i, 1), :] = row

    @pl.when(step == B // A1_ROWS - 1)
    def _():
        sim = simacc_ref[...]
        ibits = lax.bitcast_convert_type(sim, jnp.int32)
        key = jnp.where(ibits >= 0, ibits, ibits ^ POS_MASK)

        prefix = jnp.zeros((B, 1), jnp.int32)
        for b in range(31, -1, -1):
            trial = prefix | _bit(b)
            cnt = jnp.sum((key >= (trial ^ INT_MIN)).astype(jnp.float32),
                          axis=1, keepdims=True)
            prefix = jnp.where(cnt >= float(FG), trial, prefix)
        s_top = prefix ^ INT_MIN

        prefixb = jnp.zeros((B, 1), jnp.int32)
        for b in range(31, -1, -1):
            trial = prefixb | _low_mask(b)
            cnt = jnp.sum((key <= (trial ^ INT_MIN)).astype(jnp.float32),
                          axis=1, keepdims=True)
            prefixb = jnp.where(cnt >= float(BG), prefixb, prefixb | _bit(b))
        s_bot = prefixb ^ INT_MIN

        gt = key > s_top
        n_gt = jnp.sum(gt.astype(jnp.float32), axis=1, keepdims=True)
        sum_gt = jnp.sum(jnp.where(gt, sim, 0.0), axis=1, keepdims=True)
        v_top = lax.bitcast_convert_type(
            jnp.where(s_top >= 0, s_top, s_top ^ POS_MASK), jnp.float32)
        pos = (sum_gt + (FG - n_gt) * v_top) / FG

        lt = key < s_bot
        n_lt = jnp.sum(lt.astype(jnp.float32), axis=1, keepdims=True)
        sum_lt = jnp.sum(jnp.where(lt, sim, 0.0), axis=1, keepdims=True)
        v_bot = lax.bitcast_convert_type(
            jnp.where(s_bot >= 0, s_bot, s_bot ^ POS_MASK), jnp.float32)
        neg = (sum_lt + (BG - n_lt) * v_bot) / BG

        thr_ref[...] = jnp.broadcast_to(v_top, (B, 16))
        ngt_ref[...] = jnp.broadcast_to(n_gt.astype(jnp.int32), (B, 16))
        pos_ref[...] = jnp.broadcast_to(pos, (B, 16))
        neg_ref[...] = jnp.broadcast_to(neg, (B, 16))


def _run_stage_a(att_feat, aud_feat):
    aud3 = aud_feat.reshape(B, C, 1)
    return pl.pallas_call(
        _sim_topk_body,
        grid=(B // A1_ROWS,),
        in_specs=[
            pl.BlockSpec((A1_ROWS, C, K), lambda b: (b, 0, 0)),
            pl.BlockSpec((A1_ROWS, C, 1), lambda b: (b, 0, 0)),
        ],
        out_specs=[
            pl.BlockSpec((A1_ROWS, K), lambda b: (b, 0)),
            pl.BlockSpec((B, 16), lambda b: (0, 0)),
            pl.BlockSpec((B, 16), lambda b: (0, 0)),
            pl.BlockSpec((B, 16), lambda b: (0, 0)),
            pl.BlockSpec((B, 16), lambda b: (0, 0)),
        ],
        out_shape=[
            jax.ShapeDtypeStruct((B, K), jnp.float32),
            jax.ShapeDtypeStruct((B, 16), jnp.float32),
            jax.ShapeDtypeStruct((B, 16), jnp.int32),
            jax.ShapeDtypeStruct((B, 16), jnp.float32),
            jax.ShapeDtypeStruct((B, 16), jnp.float32),
        ],
        scratch_shapes=[pltpu.VMEM((B, K), jnp.float32)],
    )(att_feat, aud3)


def _sc_mask_body(sim_hbm, thr_hbm, ngt_hbm, out_hbm,
                  sim_v, thr_v, ngt_v, mask_v, sem):
    del sem
    cid = lax.axis_index("c")
    sid = lax.axis_index("s")
    wid = cid * NUM_SUBCORES + sid
    zeros16i = jnp.zeros((16,), jnp.int32)
    ones16f = jnp.full((16,), 1.0, jnp.float32)
    zeros16f = jnp.zeros((16,), jnp.float32)

    for i in range(B_PER_W):
        b = wid * B_PER_W + i
        pltpu.sync_copy(sim_hbm.at[b], sim_v)
        pltpu.sync_copy(thr_hbm.at[b], thr_v)
        pltpu.sync_copy(ngt_hbm.at[b], ngt_v)
        thr = thr_v[...]
        ngt = ngt_v[...]

        ceq = zeros16i
        for j in range(K // 16):
            svec = sim_v[pl.ds(j * 16, 16)]
            m_gt = svec > thr
            m_eq = svec == thr
            incl_e = plsc.cumsum(m_eq.astype(jnp.int32))
            tie_rank = ngt + ceq + incl_e - 1
            m_sel = jnp.logical_or(
                m_gt, jnp.logical_and(m_eq, tie_rank < FG))
            mask_v[pl.ds(j * 16, 16)] = jnp.where(m_sel, ones16f, zeros16f)
            ceq = ceq + plsc.all_reduce_population_count(m_eq)

        pltpu.sync_copy(mask_v, out_hbm.at[b])


def _run_stage_b1(sim, thr, ngt):
    mesh = plsc.VectorSubcoreMesh(core_axis_name="c", subcore_axis_name="s")
    fn = functools.partial(
        pl.kernel,
        out_type=jax.ShapeDtypeStruct((B, K), jnp.float32),
        mesh=mesh,
        compiler_params=pltpu.CompilerParams(needs_layout_passes=False),
        scratch_types=[
            pltpu.VMEM((K,), jnp.float32),
            pltpu.VMEM((16,), jnp.float32),
            pltpu.VMEM((16,), jnp.int32),
            pltpu.VMEM((K,), jnp.float32),
            pltpu.SemaphoreType.DMA,
        ],
    )(_sc_mask_body)
    return fn(sim, thr, ngt)


B2_ROWS = 4


def _hm_loss_body(hm_ref, mask_ref, pos_ref, neg_ref, av_ref,
                  dis_ref, div_ref, comb_ref):
    step = pl.program_id(0)
    for i in range(B2_ROWS):
        hm = hm_ref[i].reshape(HW, K)
        m = mask_ref[i]
        comb = lax.dot_general(m, hm, (((1,), (1,)), ((), ())),
                               preferred_element_type=jnp.float32)
        comb_ref[pl.ds(step * B2_ROWS + i, 1), :] = comb

    @pl.when(step == B // B2_ROWS - 1)
    def _():
        p = pos_ref[:, 0:1]
        n = neg_ref[:, 0:1]
        mx = jnp.maximum(p, n)
        lse = mx + jnp.log(jnp.exp(p - mx) + jnp.exp(n - mx))
        dis_ref[...] = jnp.mean(lse - p).reshape(1, 1)

        c = comb_ref[...] * (1.0 / FG)
        a = av_ref[...]
        cm = jnp.max(c, axis=1, keepdims=True)
        ce = jnp.exp(c - cm)
        cz = jnp.sum(ce, axis=1, keepdims=True)
        att = ce / cz
        log_att = (c - cm) - jnp.log(cz)
        am = jnp.max(a, axis=1, keepdims=True)
        ae = jnp.exp(a - am)
        az = jnp.sum(ae, axis=1, keepdims=True)
        avd = ae / az
        log_av = (a - am) - jnp.log(az)
        logm = jnp.log(0.5 * (att + avd))
        div = (jnp.sum(att * (log_att - logm)) +
               jnp.sum(avd * (log_av - logm))) / (2.0 * B)
        div_ref[...] = div.reshape(1, 1)


def _run_stage_b2c(hm_t, mask, pos, neg, av):
    return pl.pallas_call(
        _hm_loss_body,
        grid=(B // B2_ROWS,),
        in_specs=[
            pl.BlockSpec((B2_ROWS, H, W, K), lambda b: (b, 0, 0, 0)),
            pl.BlockSpec((B2_ROWS, 1, K), lambda b: (b, 0, 0)),
            pl.BlockSpec((B, 16), lambda b: (0, 0)),
            pl.BlockSpec((B, 16), lambda b: (0, 0)),
            pl.BlockSpec((B, HW), lambda b: (0, 0)),
        ],
        out_specs=[
            pl.BlockSpec((1, 1), lambda b: (0, 0)),
            pl.BlockSpec((1, 1), lambda b: (0, 0)),
        ],
        out_shape=[
            jax.ShapeDtypeStruct((1, 1), jnp.float32),
            jax.ShapeDtypeStruct((1, 1), jnp.float32),
        ],
        scratch_shapes=[pltpu.VMEM((B, HW), jnp.float32)],
    )(hm_t, mask.reshape(B, 1, K), pos, neg, av)


def kernel(att_feat, aud_feat, att_heatmaps, av_heatmaps):
    sim, thr, ngt, pos, neg = _run_stage_a(att_feat, aud_feat)
    mask = _run_stage_b1(sim, thr, ngt)
    hm_t = jnp.transpose(att_heatmaps, (0, 2, 3, 1))
    dis, div = _run_stage_b2c(hm_t, mask, pos, neg,
                              av_heatmaps.reshape(B, HW))
    return dis.reshape(()), div.reshape(())
